# baseline (speedup 1.0000x reference)
import functools

import jax
import jax.numpy as jnp
from jax import lax
from jax.experimental import pallas as pl
from jax.experimental.pallas import tpu as pltpu
from jax.experimental.pallas import tpu_sc as plsc

NUM_EMB = 1_000_000
DIM = 64
PDIM = 128
ROWS = 16384
COLS = 26
SLOTS = 32

NC = 2
NS = 16
NW = NC * NS

L = 16

_mesh = plsc.VectorSubcoreMesh(core_axis_name="c", subcore_axis_name="s")

U = 128
N_UNITS = NUM_EMB // U
TAIL = NUM_EMB - N_UNITS * U
NK = -(-N_UNITS // NW)

assert NK % 2 == 1


@functools.partial(
    pl.kernel,
    mesh=_mesh,
    out_type=jax.ShapeDtypeStruct((NUM_EMB, PDIM), jnp.float32),
    scratch_types=[
        pltpu.VMEM((DIM, U), jnp.float32),
        pltpu.VMEM((DIM, U), jnp.float32),
        pltpu.VMEM((U, PDIM), jnp.float32),
        pltpu.VMEM((U, PDIM), jnp.float32),
        pltpu.VMEM((TAIL, DIM), jnp.float32),
        pltpu.SemaphoreType.DMA,
        pltpu.SemaphoreType.DMA,
        pltpu.SemaphoreType.DMA,
        pltpu.SemaphoreType.DMA,
    ],
    compiler_params=pltpu.CompilerParams(needs_layout_passes=False),
)
def _transpose_sc(wt_hbm, tail_hbm, table_hbm, a0, a1, b0, b1, t_v,
                  la0, la1, sb0, sb1):
    wid = lax.axis_index("s") * NC + lax.axis_index("c")

    cols16 = [lax.iota(jnp.int32, L) + L * j for j in range(DIM // L)]

    def uof(k):
        return jnp.minimum(wid + NW * k, N_UNITS - 1)

    def load_a(k, av, sem):
        pltpu.async_copy(wt_hbm.at[:, pl.ds(uof(k) * U, U)], av, sem)

    def wait_a(av, sem):
        pltpu.make_async_copy(wt_hbm.at[:, pl.ds(0, U)], av, sem).wait()

    def store_b(k, bv, sem):
        pltpu.async_copy(bv, table_hbm.at[pl.ds(uof(k) * U, U)], sem)

    def wait_b(bv, sem):
        pltpu.make_async_copy(bv, table_hbm.at[pl.ds(0, U)], sem).wait()

    def transpose(av, bv):
        for rr in range(U):
            row = jnp.full((L,), rr, jnp.int32)
            for j in range(DIM // L):
                bv[rr, pl.ds(L * j, L)] = plsc.load_gather(
                    av, [cols16[j], row])

    load_a(0, a0, la0)
    wait_a(a0, la0)
    load_a(1, a1, la1)
    transpose(a0, b0)
    store_b(0, b0, sb0)
    wait_a(a1, la1)
    load_a(2, a0, la0)
    transpose(a1, b1)
    store_b(1, b1, sb1)

    def body(K, _):
        k2 = 2 * K + 2
        wait_a(a0, la0)
        load_a(k2 + 1, a1, la1)
        wait_b(b0, sb0)
        transpose(a0, b0)
        store_b(k2, b0, sb0)
        k3 = 2 * K + 3
        wait_a(a1, la1)
        load_a(k3 + 1, a0, la0)
        wait_b(b1, sb1)
        transpose(a1, b1)
        store_b(k3, b1, sb1)
        return 0

    lax.fori_loop(0, (NK - 3) // 2, body, 0)

    wait_a(a0, la0)
    wait_b(b0, sb0)
    transpose(a0, b0)
    store_b(NK - 1, b0, sb0)

    @pl.when(wid == NW - 1)
    def _():
        wait_b(b1, sb1)
        pltpu.sync_copy(tail_hbm, t_v)
        for rr in range(TAIL):
            for j in range(DIM // L):
                b1[rr, pl.ds(L * j, L)] = t_v[rr, pl.ds(L * j, L)]
        pltpu.async_copy(b1.at[pl.ds(0, TAIL)],
                         table_hbm.at[pl.ds(N_UNITS * U, TAIL)], sb1)
        pltpu.make_async_copy(b1.at[pl.ds(0, TAIL)],
                              table_hbm.at[pl.ds(0, TAIL)], sb1).wait()

    @pl.when(wid != NW - 1)
    def _():
        wait_b(b1, sb1)

    wait_b(b0, sb0)


CB = 8
R_PER_W = ROWS // NW
N_CHUNKS = R_PER_W // CB
CROWS = CB * SLOTS

assert R_PER_W % CB == 0 and N_CHUNKS % 2 == 0


@functools.partial(
    pl.kernel,
    mesh=_mesh,
    out_type=jax.ShapeDtypeStruct((ROWS * SLOTS, PDIM), jnp.float32),
    scratch_types=[
        pltpu.VMEM((R_PER_W, COLS), jnp.int32),
        pltpu.VMEM((CROWS, PDIM), jnp.float32),
        pltpu.VMEM((CROWS, PDIM), jnp.float32),
        pltpu.SemaphoreType.DMA,
        pltpu.SemaphoreType.DMA,
        pltpu.SemaphoreType.DMA,
    ],
)
def _emb_lookup(idx_hbm, table_hbm, out_hbm, idx_v, rows0, rows1, gsem,
                osem0, osem1):
    wid = lax.axis_index("s") * NC + lax.axis_index("c")
    row0 = wid * R_PER_W

    def gather(i, rbuf):
        for blk in range(CB):
            pltpu.async_copy(
                table_hbm.at[idx_v.at[i * CB + blk]],
                rbuf.at[pl.ds(blk * SLOTS, COLS)],
                gsem,
            )

    def wait_gather(rbuf):
        for blk in range(CB):
            pltpu.make_async_copy(
                table_hbm.at[idx_v.at[blk]],
                rbuf.at[pl.ds(blk * SLOTS, COLS)],
                gsem,
            ).wait()

    def store(i, rbuf, osem):
        pltpu.async_copy(
            rbuf, out_hbm.at[pl.ds((row0 + i * CB) * SLOTS, CROWS)], osem)

    def wait_store(rbuf, osem):
        pltpu.make_async_copy(
            rbuf, out_hbm.at[pl.ds(0, CROWS)], osem).wait()

    pltpu.sync_copy(idx_hbm.at[pl.ds(row0, R_PER_W)], idx_v)

    gather(0, rows0)
    wait_gather(rows0)
    gather(1, rows1)
    store(0, rows0, osem0)

    def body(k, _):
        i1 = 2 * k + 1
        wait_gather(rows1)
        wait_store(rows0, osem0)
        gather(i1 + 1, rows0)
        store(i1, rows1, osem1)
        i2 = 2 * k + 2
        wait_gather(rows0)
        wait_store(rows1, osem1)
        gather(i2 + 1, rows1)
        store(i2, rows0, osem0)
        return 0

    lax.fori_loop(0, N_CHUNKS // 2 - 1, body, 0)

    wait_gather(rows1)
    wait_store(rows0, osem0)
    store(N_CHUNKS - 1, rows1, osem1)
    wait_store(rows1, osem1)


def kernel(x, Weights):
    table = _transpose_sc(Weights.T, Weights[N_UNITS * U:])
    out = _emb_lookup(x.astype(jnp.int32), table)
    return out.reshape(ROWS, SLOTS, PDIM)[:, :COLS, :DIM]


# speedup vs baseline: 2.3294x; 2.3294x over previous
import functools

import jax
import jax.numpy as jnp
from jax import lax
from jax.experimental import pallas as pl
from jax.experimental.pallas import tpu as pltpu
from jax.experimental.pallas import tpu_sc as plsc

NUM_EMB = 1_000_000
DIM = 64
PDIM = 128
ROWS = 16384
COLS = 26
SLOTS = 32

NC = 2
NS = 16
NW = NC * NS

L = 16

_mesh = plsc.VectorSubcoreMesh(core_axis_name="c", subcore_axis_name="s")

CB = 8
R_PER_W = ROWS // NW
N_CHUNKS = R_PER_W // CB
CROWS = CB * SLOTS

assert R_PER_W % CB == 0 and N_CHUNKS % 2 == 0


@functools.partial(
    pl.kernel,
    mesh=_mesh,
    out_type=jax.ShapeDtypeStruct((ROWS * SLOTS, PDIM), jnp.float32),
    scratch_types=[
        pltpu.VMEM((R_PER_W, COLS), jnp.int32),
        pltpu.VMEM((CROWS, PDIM), jnp.float32),
        pltpu.VMEM((CROWS, PDIM), jnp.float32),
        pltpu.SemaphoreType.DMA,
        pltpu.SemaphoreType.DMA,
        pltpu.SemaphoreType.DMA,
    ],
)
def _emb_lookup(idx_hbm, table_hbm, out_hbm, idx_v, rows0, rows1, gsem,
                osem0, osem1):
    wid = lax.axis_index("s") * NC + lax.axis_index("c")
    row0 = wid * R_PER_W

    def gather(i, rbuf):
        for blk in range(CB):
            pltpu.async_copy(
                table_hbm.at[idx_v.at[i * CB + blk]],
                rbuf.at[pl.ds(blk * SLOTS, COLS)],
                gsem,
            )

    def wait_gather(rbuf):
        for blk in range(CB):
            pltpu.make_async_copy(
                table_hbm.at[idx_v.at[blk]],
                rbuf.at[pl.ds(blk * SLOTS, COLS)],
                gsem,
            ).wait()

    def store(i, rbuf, osem):
        pltpu.async_copy(
            rbuf, out_hbm.at[pl.ds((row0 + i * CB) * SLOTS, CROWS)], osem)

    def wait_store(rbuf, osem):
        pltpu.make_async_copy(
            rbuf, out_hbm.at[pl.ds(0, CROWS)], osem).wait()

    pltpu.sync_copy(idx_hbm.at[pl.ds(row0, R_PER_W)], idx_v)

    gather(0, rows0)
    wait_gather(rows0)
    gather(1, rows1)
    store(0, rows0, osem0)

    def body(k, _):
        i1 = 2 * k + 1
        wait_gather(rows1)
        wait_store(rows0, osem0)
        gather(i1 + 1, rows0)
        store(i1, rows1, osem1)
        i2 = 2 * k + 2
        wait_gather(rows0)
        wait_store(rows1, osem1)
        gather(i2 + 1, rows1)
        store(i2, rows0, osem0)
        return 0

    lax.fori_loop(0, N_CHUNKS // 2 - 1, body, 0)

    wait_gather(rows1)
    wait_store(rows0, osem0)
    store(N_CHUNKS - 1, rows1, osem1)
    wait_store(rows1, osem1)


def kernel(x, Weights):
    table = jnp.pad(Weights, ((0, 0), (0, PDIM - DIM)))
    out = _emb_lookup(x.astype(jnp.int32), table)
    return out.reshape(ROWS, SLOTS, PDIM)[:, :COLS, :DIM]
